# baseline (speedup 1.0000x reference)
import functools

import jax
import jax.numpy as jnp
from jax import lax
from jax.experimental import pallas as pl
from jax.experimental.pallas import tpu as pltpu
from jax.experimental.pallas import tpu_sc as plsc

_BETA = 0.25
_CODE_CHUNK = 4096
_ROW_TILE = 512


def _assign_body(inv_nd, z_ref, cbt_ref, esq_ref, idx_ref, loss_ref, acc_ref):
    z = z_ref[...]
    r = z.shape[0]
    ncodes = cbt_ref.shape[1]
    z_sq = jnp.sum(z * z, axis=1, keepdims=True)

    run_min = jnp.full((r, 1), jnp.inf, dtype=jnp.float32)
    run_idx = jnp.zeros((r, 1), dtype=jnp.int32)
    true_min = jnp.full((r, 1), jnp.inf, dtype=jnp.float32)
    lane_f = lax.broadcasted_iota(
        jnp.int32, (r, _CODE_CHUNK), 1).astype(jnp.float32)
    for c in range(ncodes // _CODE_CHUNK):
        cbt_c = cbt_ref[:, pl.ds(c * _CODE_CHUNK, _CODE_CHUNK)]
        e_sq = esq_ref[:, pl.ds(c * _CODE_CHUNK, _CODE_CHUNK)]
        ze = lax.dot_general(z, cbt_c, (((1,), (0,)), ((), ())),
                             preferred_element_type=jnp.float32)
        dist = (z_sq + ze) + e_sq
        m = jnp.min(dist, axis=1, keepdims=True)
        i_f = jnp.min(jnp.where(dist == m, lane_f, float(_CODE_CHUNK)),
                      axis=1, keepdims=True)
        i_c = i_f.astype(jnp.int32) + c * _CODE_CHUNK
        keep = (run_min < m) | ((run_min == m) & (run_idx < i_c))
        run_idx = jnp.where(keep, run_idx, i_c)
        sel = jnp.where(keep, run_min, m)
        run_min = sel.astype(jnp.bfloat16).astype(jnp.float32)
        true_min = jnp.minimum(true_min, m)

    idx_ref[...] = run_idx
    step = pl.program_id(0)

    @pl.when(step == 0)
    def _():
        acc_ref[0] = 0.0

    acc_ref[0] += jnp.sum(true_min)

    @pl.when(step == pl.num_programs(0) - 1)
    def _():
        loss_ref[...] = jnp.broadcast_to(
            acc_ref[0] * ((1.0 + _BETA) * inv_nd), (1, 1))


def _assign(z_flat, cbt, e_sq):
    n_rows, d = z_flat.shape
    ncodes = cbt.shape[1]
    body = functools.partial(_assign_body, 1.0 / float(n_rows * d))
    idx, loss = pl.pallas_call(
        body,
        grid=(n_rows // _ROW_TILE,),
        in_specs=[
            pl.BlockSpec((_ROW_TILE, d), lambda i: (i, 0)),
            pl.BlockSpec((d, ncodes), lambda i: (0, 0)),
            pl.BlockSpec((1, ncodes), lambda i: (0, 0)),
        ],
        out_specs=[
            pl.BlockSpec((_ROW_TILE, 1), lambda i: (i, 0)),
            pl.BlockSpec((1, 1), lambda i: (0, 0)),
        ],
        out_shape=[
            jax.ShapeDtypeStruct((n_rows, 1), jnp.int32),
            jax.ShapeDtypeStruct((1, 1), jnp.float32),
        ],
        scratch_shapes=[pltpu.SMEM((1,), jnp.float32)],
    )(z_flat, cbt, e_sq)
    return idx.reshape(n_rows), loss.reshape(())


def _gather_zq(codebook, idx_flat):
    b_total = idx_flat.shape[0]
    d = codebook.shape[1]
    info = plsc.get_sparse_core_info()
    nc = info.num_cores
    nw = nc * info.num_subcores
    chunk = 128
    n_chunks = b_total // (nw * chunk)
    mesh = plsc.VectorSubcoreMesh(core_axis_name="c", subcore_axis_name="s")

    @functools.partial(
        pl.kernel,
        mesh=mesh,
        compiler_params=pltpu.CompilerParams(use_tc_tiling_on_sc=False),
        out_type=jax.ShapeDtypeStruct((b_total, d), jnp.float32),
        scratch_types=[
            pltpu.VMEM((n_chunks, chunk), jnp.int32),
            pltpu.VMEM((chunk, d), jnp.float32),
            pltpu.SemaphoreType.DMA,
        ],
    )
    def gather_k(table_hbm, idx_hbm, out_hbm, idx_v, rows_v, sem):
        wid = lax.axis_index("s") * nc + lax.axis_index("c")
        pltpu.sync_copy(idx_hbm.at[pl.ds(wid * n_chunks, n_chunks)], idx_v)
        for j in range(n_chunks):
            pltpu.async_copy(table_hbm.at[idx_v.at[j]], rows_v, sem).wait()
            pltpu.sync_copy(
                rows_v, out_hbm.at[pl.ds((wid * n_chunks + j) * chunk, chunk)])

    return gather_k(codebook, idx_flat.reshape(b_total // chunk, chunk))


def kernel(z_e, codebook):
    b, n, d = z_e.shape
    z_flat = z_e.reshape(b * n, d)
    e_sq = jnp.sum(codebook * codebook, axis=1, keepdims=True).T
    idx_flat, vq_loss = _assign(z_flat, (-2.0 * codebook).T, e_sq)
    z_q = _gather_zq(codebook, idx_flat).reshape(b, n, d)
    z_q_st = z_e + lax.stop_gradient(z_q - z_e)
    return (z_q_st, vq_loss, idx_flat.reshape(b, n))


# speedup vs baseline: 1.0274x; 1.0274x over previous
import functools

import jax
import jax.numpy as jnp
from jax import lax
from jax.experimental import pallas as pl
from jax.experimental.pallas import tpu as pltpu
from jax.experimental.pallas import tpu_sc as plsc

_BETA = 0.25
_CODE_CHUNK = 4096
_ROW_TILE = 512


def _assign_body(inv_nd, z_ref, cbt_ref, esq_ref, idx_ref, loss_ref, acc_ref):
    z = z_ref[...]
    r = z.shape[0]
    ncodes = cbt_ref.shape[1]
    z_sq = jnp.sum(z * z, axis=1, keepdims=True)

    run_min = jnp.full((r, 1), jnp.inf, dtype=jnp.float32)
    run_idx = jnp.zeros((r, 1), dtype=jnp.int32)
    true_min = jnp.full((r, 1), jnp.inf, dtype=jnp.float32)
    lane_f = lax.broadcasted_iota(
        jnp.int32, (r, _CODE_CHUNK), 1).astype(jnp.float32)
    for c in range(ncodes // _CODE_CHUNK):
        cbt_c = cbt_ref[:, pl.ds(c * _CODE_CHUNK, _CODE_CHUNK)]
        e_sq = esq_ref[:, pl.ds(c * _CODE_CHUNK, _CODE_CHUNK)]
        ze = lax.dot_general(z, cbt_c, (((1,), (0,)), ((), ())),
                             preferred_element_type=jnp.float32)
        dist = (z_sq - 2.0 * ze) + e_sq
        m = jnp.min(dist, axis=1, keepdims=True)
        i_f = jnp.min(jnp.where(dist == m, lane_f, float(_CODE_CHUNK)),
                      axis=1, keepdims=True)
        i_c = i_f.astype(jnp.int32) + c * _CODE_CHUNK
        keep = (run_min < m) | ((run_min == m) & (run_idx < i_c))
        run_idx = jnp.where(keep, run_idx, i_c)
        sel = jnp.where(keep, run_min, m)
        run_min = sel.astype(jnp.bfloat16).astype(jnp.float32)
        true_min = jnp.minimum(true_min, m)

    idx_ref[...] = run_idx
    step = pl.program_id(0)

    @pl.when(step == 0)
    def _():
        acc_ref[0] = 0.0

    acc_ref[0] += jnp.sum(true_min)

    @pl.when(step == pl.num_programs(0) - 1)
    def _():
        loss_ref[...] = jnp.broadcast_to(
            acc_ref[0] * ((1.0 + _BETA) * inv_nd), (1, 1))


def _assign(z_flat, cbt, e_sq):
    n_rows, d = z_flat.shape
    ncodes = cbt.shape[1]
    body = functools.partial(_assign_body, 1.0 / float(n_rows * d))
    idx, loss = pl.pallas_call(
        body,
        grid=(n_rows // _ROW_TILE,),
        in_specs=[
            pl.BlockSpec((_ROW_TILE, d), lambda i: (i, 0)),
            pl.BlockSpec((d, ncodes), lambda i: (0, 0)),
            pl.BlockSpec((1, ncodes), lambda i: (0, 0)),
        ],
        out_specs=[
            pl.BlockSpec((_ROW_TILE, 1), lambda i: (i, 0)),
            pl.BlockSpec((1, 1), lambda i: (0, 0)),
        ],
        out_shape=[
            jax.ShapeDtypeStruct((n_rows, 1), jnp.int32),
            jax.ShapeDtypeStruct((1, 1), jnp.float32),
        ],
        scratch_shapes=[pltpu.SMEM((1,), jnp.float32)],
    )(z_flat, cbt, e_sq)
    return idx.reshape(n_rows), loss.reshape(())


def _gather_zq(codebook, idx_flat):
    b_total = idx_flat.shape[0]
    d = codebook.shape[1]
    info = plsc.get_sparse_core_info()
    nc = info.num_cores
    nw = nc * info.num_subcores
    chunk = 128
    n_chunks = b_total // (nw * chunk)
    mesh = plsc.VectorSubcoreMesh(core_axis_name="c", subcore_axis_name="s")

    @functools.partial(
        pl.kernel,
        mesh=mesh,
        compiler_params=pltpu.CompilerParams(use_tc_tiling_on_sc=False),
        out_type=jax.ShapeDtypeStruct((b_total, d), jnp.float32),
        scratch_types=[
            pltpu.VMEM((n_chunks, chunk), jnp.int32),
            pltpu.VMEM((chunk, d), jnp.float32),
            pltpu.SemaphoreType.DMA,
        ],
    )
    def gather_k(table_hbm, idx_hbm, out_hbm, idx_v, rows_v, sem):
        wid = lax.axis_index("s") * nc + lax.axis_index("c")
        pltpu.sync_copy(idx_hbm.at[pl.ds(wid * n_chunks, n_chunks)], idx_v)
        for j in range(n_chunks):
            pltpu.async_copy(table_hbm.at[idx_v.at[j]], rows_v, sem).wait()
            pltpu.sync_copy(
                rows_v, out_hbm.at[pl.ds((wid * n_chunks + j) * chunk, chunk)])

    return gather_k(codebook, idx_flat.reshape(b_total // chunk, chunk))


def kernel(z_e, codebook):
    b, n, d = z_e.shape
    z_flat = z_e.reshape(b * n, d)
    e_sq = jnp.sum(codebook * codebook, axis=1, keepdims=True).T
    idx_flat, vq_loss = _assign(z_flat, codebook.T, e_sq)
    z_q = _gather_zq(codebook, idx_flat).reshape(b, n, d)
    z_q_st = z_e + lax.stop_gradient(z_q - z_e)
    return (z_q_st, vq_loss, idx_flat.reshape(b, n))
